# baseline (speedup 1.0000x reference)
import functools

import jax
import jax.numpy as jnp
import numpy as np
from jax import lax
from jax.experimental import pallas as pl
from jax.experimental.pallas import tpu as pltpu
from jax.experimental.pallas import tpu_sc as plsc

VOXEL_SIZE = np.array([0.4, 0.4, 8.0], dtype=np.float32)
PC_RANGE = np.array([-51.2, -51.2, -5.0, 51.2, 51.2, 3.0], dtype=np.float32)
BATCH = 2
NSWEEPS = 2
GX = 256
GY = 256
D_OUT = 64
BN_EPS = 1e-3
N_RAW = 200000

NSEG = BATCH * NSWEEPS * GY * GX
NPAD = 200704
N_TILES = 32
PPT = NPAD // N_TILES
NREG = 256
NREGP = 272
ROWS_PER_REG = 1024
REG_PER_TILE = NREG // N_TILES
BIN_ROWS = NPAD + NREGP * 8 + 512
MEAN_ROWS = NSEG + 1024
CHUNK = 256

_MESH = dict(core_axis_name="c", subcore_axis_name="s")
_SC_PARAMS = pltpu.CompilerParams(use_tc_tiling_on_sc=False, needs_layout_passes=False)

_KM_PTBODY = False
_KM_SHADOW = True
_KM_STATIC_NCH = True
_KM_CHUNKS = False
_USE_SC_BIN = True
_USE_SC_MEAN = True
_USE_SC_MAX = False
_USE_TC_PRE = True


def _perm16(x, idx):
    dn = lax.GatherDimensionNumbers(
        offset_dims=(), collapsed_slice_dims=(0,), start_index_map=(0,)
    )
    return lax.gather(
        x, idx[:, None], dn, (1,),
        mode=lax.GatherScatterMode.PROMISE_IN_BOUNDS,
    )


def _scalar_at(ref, i):
    ib = pl.multiple_of((i >> 4) << 4, 16)
    v = ref[pl.ds(ib, 16)]
    lane = jnp.zeros((16,), jnp.int32) + (i & 15)
    return _perm16(v, lane)[0]


@functools.partial(
    pl.kernel,
    mesh=plsc.VectorSubcoreMesh(**_MESH),
    compiler_params=_SC_PARAMS,
    out_type=(
        jax.ShapeDtypeStruct((N_TILES * NREGP,), jnp.int32),
        jax.ShapeDtypeStruct((NPAD,), jnp.int32),
    ),
    scratch_types=[
        pltpu.VMEM((PPT,), jnp.int32),
        pltpu.VMEM((PPT,), jnp.int32),
        pltpu.VMEM((NREGP,), jnp.int32),
    ],
)
def _k2b1_hist_rank(seg_hbm, hist_hbm, rank_hbm, seg_v, rank_v, cnt_v):
    c = lax.axis_index("c")
    s = lax.axis_index("s")
    wid = c * 16 + s
    base = wid * PPT
    pltpu.sync_copy(seg_hbm.at[pl.ds(base, PPT)], seg_v)
    zeros16 = jnp.zeros((16,), jnp.int32)
    for k in range(NREGP // 16):
        cnt_v[pl.ds(k * 16, 16)] = zeros16
    lanes = lax.iota(jnp.int32, 16)
    prev_l = jnp.maximum(lanes - 1, 0)
    next_l = jnp.minimum(lanes + 1, 15)

    def body(k, carry):
        sv = seg_v[pl.ds(k * 16, 16)]
        r = sv >> 10
        sr, slane = plsc.sort_key_val(r, lanes)
        shifted = _perm16(sr, prev_l)
        is_new = (lanes == 0) | (sr != shifted)
        start_pos = plsc.cummax(jnp.where(is_new, lanes, 0))
        rank_in = lanes - start_pos
        cbase = plsc.load_gather(cnt_v, [sr])
        nxt_new = _perm16(is_new.astype(jnp.int32), next_l)
        is_last = (lanes == 15) | (nxt_new == 1)
        plsc.store_scatter(cnt_v, [sr], cbase + rank_in + 1, mask=is_last)
        plsc.store_scatter(rank_v, [k * 16 + slane], cbase + rank_in)
        return carry

    lax.fori_loop(0, PPT // 16, body, 0)
    pltpu.sync_copy(rank_v, rank_hbm.at[pl.ds(base, PPT)])
    pltpu.sync_copy(cnt_v, hist_hbm.at[pl.ds(wid * NREGP, NREGP)])


def _k2b2_body(hist_ref, offw_ref, goff_ref, cnts_ref):
    h = hist_ref[...].astype(jnp.float32)
    iw = lax.broadcasted_iota(jnp.int32, (N_TILES, N_TILES), 0)
    jw = lax.broadcasted_iota(jnp.int32, (N_TILES, N_TILES), 1)
    tri_w = (iw > jw).astype(jnp.float32)
    s_w = jax.lax.dot_general(
        tri_w, h, (((1,), (0,)), ((), ())),
        precision=jax.lax.Precision.HIGHEST,
    )
    tot = jnp.sum(h, axis=0)
    tot8 = jnp.ceil(tot / 8.0) * 8.0
    ir = lax.broadcasted_iota(jnp.int32, (NREGP, NREGP), 0)
    jr = lax.broadcasted_iota(jnp.int32, (NREGP, NREGP), 1)
    tri_r = (ir > jr).astype(jnp.float32)
    g = jax.lax.dot_general(
        tri_r, tot8[:, None], (((1,), (0,)), ((), ())),
        precision=jax.lax.Precision.HIGHEST,
    )[:, 0]
    offw_ref[...] = (g[None, :] + s_w).astype(jnp.int32)
    goff_ref[...] = g.astype(jnp.int32)
    cnts_ref[...] = tot.astype(jnp.int32)


def _k2b2_offsets(hist):
    return pl.pallas_call(
        _k2b2_body,
        out_shape=(
            jax.ShapeDtypeStruct((N_TILES, NREGP), jnp.int32),
            jax.ShapeDtypeStruct((NREGP,), jnp.int32),
            jax.ShapeDtypeStruct((NREGP,), jnp.int32),
        ),
    )(hist)


@functools.partial(
    pl.kernel,
    mesh=plsc.VectorSubcoreMesh(**_MESH),
    compiler_params=_SC_PARAMS,
    out_type=jax.ShapeDtypeStruct((BIN_ROWS,), jnp.int32),
    scratch_types=[
        pltpu.VMEM((PPT,), jnp.int32),
        pltpu.VMEM((PPT,), jnp.int32),
        pltpu.VMEM((NREGP,), jnp.int32),
        pltpu.VMEM((PPT,), jnp.int32),
        pltpu.VMEM((PPT,), jnp.int32),
    ],
)
def _k2b3_place(seg_hbm, rank_hbm, offw_hbm, binned_hbm,
                seg_v, rank_v, off_v, dest_v, packed_v):
    c = lax.axis_index("c")
    s = lax.axis_index("s")
    wid = c * 16 + s
    base = wid * PPT
    pltpu.sync_copy(seg_hbm.at[pl.ds(base, PPT)], seg_v)
    pltpu.sync_copy(rank_hbm.at[pl.ds(base, PPT)], rank_v)
    pltpu.sync_copy(offw_hbm.at[pl.ds(wid * NREGP, NREGP)], off_v)
    lanes = lax.iota(jnp.int32, 16)

    def body(k, carry):
        sl = pl.ds(k * 16, 16)
        sv = seg_v[sl]
        r = sv >> 10
        off = plsc.load_gather(off_v, [r])
        dest_v[sl] = off + rank_v[sl]
        pid = base + k * 16 + lanes
        packed_v[sl] = pid * 1024 + (sv & 1023)
        return carry

    lax.fori_loop(0, PPT // 16, body, 0)
    pltpu.sync_copy(packed_v, binned_hbm.at[dest_v])


@functools.partial(
    pl.kernel,
    mesh=plsc.VectorSubcoreMesh(**_MESH),
    compiler_params=_SC_PARAMS,
    out_type=jax.ShapeDtypeStruct((MEAN_ROWS, 16), jnp.float32),
    scratch_types=[
        pltpu.VMEM((ROWS_PER_REG, 16), jnp.float32),
        pltpu.VMEM((CHUNK,), jnp.int32),
        pltpu.VMEM((CHUNK,), jnp.int32),
        pltpu.VMEM((CHUNK + 16,), jnp.int32),
        pltpu.VMEM((CHUNK, 16), jnp.float32),
        pltpu.VMEM((NREGP,), jnp.int32),
        pltpu.VMEM((NREGP,), jnp.int32),
        pltpu.SemaphoreType.DMA,
    ],
)
def _km_means(binned_hbm, xyzv_hbm, goff_hbm, cnts_hbm, mean_hbm,
              acc_v, chunk_v, ids_v, rows_v, xrows_v, goff_v, cnts_v, sem):
    c = lax.axis_index("c")
    s = lax.axis_index("s")
    wid = c * 16 + s
    pltpu.sync_copy(goff_hbm, goff_v)
    pltpu.sync_copy(cnts_hbm, cnts_v)
    lanes = lax.iota(jnp.int32, 16)
    addmask = lanes < 4
    zeros16 = jnp.zeros((16,), jnp.float32)

    for p in range(REG_PER_TILE):
        r = wid * REG_PER_TILE + p
        n0 = _scalar_at(goff_v, r)
        cnt = _scalar_at(cnts_v, r)

        def zbody(i, carry):
            acc_v[i, :] = zeros16
            return carry

        lax.fori_loop(0, ROWS_PER_REG, zbody, 0)

        def chunk_body(k, carry):
            j0 = pl.multiple_of(n0 + k * CHUNK, 8)
            pltpu.sync_copy(binned_hbm.at[pl.ds(j0, CHUNK)], chunk_v)
            for kk in range(CHUNK // 16):
                sl = pl.ds(kk * 16, 16)
                v = chunk_v[sl]
                ids_v[sl] = jnp.minimum(v >> 10, NPAD - 1)
                rows_v[sl] = v & 1023
            pltpu.async_copy(xyzv_hbm.at[ids_v], xrows_v, sem).wait()
            cc = jnp.minimum(cnt - k * CHUNK, CHUNK)

            if _KM_PTBODY:
                def pt_body(i, carry2):
                    row = _scalar_at(rows_v, i)
                    xr = xrows_v[i, :]
                    acc_v[row, :] = acc_v[row, :] + jnp.where(addmask, xr, 0.0)
                    return carry2

                lax.fori_loop(0, cc, pt_body, 0)
            return carry

        nch = (cnt + CHUNK - 1) // CHUNK
        if _KM_CHUNKS:
            if _KM_STATIC_NCH:
                lax.fori_loop(0, 3, chunk_body, 0)
            else:
                lax.fori_loop(0, nch, chunk_body, 0)

            def mean_body(i, carry):
                v = acc_v[i, :]
                n = jnp.max(jnp.where(lanes == 3, v, 0.0))
                acc_v[i, :] = v / jnp.maximum(n, 1.0)
                return carry

            lax.fori_loop(0, ROWS_PER_REG, mean_body, 0)
        pltpu.sync_copy(acc_v, mean_hbm.at[pl.ds(r * ROWS_PER_REG, ROWS_PER_REG)])


@functools.partial(
    pl.kernel,
    mesh=plsc.VectorSubcoreMesh(**_MESH),
    compiler_params=_SC_PARAMS,
    out_type=jax.ShapeDtypeStruct((NPAD, 16), jnp.float32),
    scratch_types=[
        pltpu.VMEM((PPT,), jnp.int32),
        pltpu.VMEM((PPT, 16), jnp.float32),
        pltpu.SemaphoreType.DMA,
    ],
)
def _k4_gather_mean(mean_hbm, seg_hbm, out_hbm, idx_v, rows_v, sem):
    c = lax.axis_index("c")
    s = lax.axis_index("s")
    base = (c * 16 + s) * PPT
    pltpu.sync_copy(seg_hbm.at[pl.ds(base, PPT)], idx_v)
    pltpu.async_copy(mean_hbm.at[idx_v], rows_v, sem).wait()
    pltpu.sync_copy(rows_v, out_hbm.at[pl.ds(base, PPT)])


@functools.partial(
    pl.kernel,
    mesh=plsc.VectorSubcoreMesh(**_MESH),
    compiler_params=_SC_PARAMS,
    out_type=jax.ShapeDtypeStruct((BATCH, D_OUT, NSWEEPS, 64, 1024), jnp.float32),
    scratch_types=[
        pltpu.VMEM((D_OUT, ROWS_PER_REG), jnp.float32),
        pltpu.VMEM((CHUNK,), jnp.int32),
        pltpu.VMEM((CHUNK,), jnp.int32),
        pltpu.VMEM((CHUNK + 16,), jnp.int32),
        pltpu.VMEM((CHUNK, D_OUT), jnp.float32),
        pltpu.VMEM((NREGP,), jnp.int32),
        pltpu.VMEM((NREGP,), jnp.int32),
        pltpu.SemaphoreType.DMA,
    ],
)
def _k8_scatter_max(binned_hbm, x_hbm, goff_hbm, cnts_hbm, out_hbm,
                    canvas_v, chunk_v, ids_v, rows_v, xrows_v,
                    goff_v, cnts_v, sem):
    c = lax.axis_index("c")
    s = lax.axis_index("s")
    wid = c * 16 + s
    pltpu.sync_copy(goff_hbm, goff_v)
    pltpu.sync_copy(cnts_hbm, cnts_v)
    lanes = lax.iota(jnp.int32, 16)
    featv = [lanes + 16 * cc for cc in range(4)]
    zeros16 = jnp.zeros((16,), jnp.float32)

    for p in range(REG_PER_TILE):
        r = wid * REG_PER_TILE + p
        b = r >> 7
        t = (r >> 6) & 1
        cy4 = r & 63
        n0 = _scalar_at(goff_v, r)
        cnt = _scalar_at(cnts_v, r)

        def zbody(i, carry):
            def zinner(k, carry2):
                canvas_v[i, pl.ds(k * 16, 16)] = zeros16
                return carry2
            lax.fori_loop(0, ROWS_PER_REG // 16, zinner, 0)
            return carry

        lax.fori_loop(0, D_OUT, zbody, 0)

        def chunk_body(k, carry):
            j0 = pl.multiple_of(n0 + k * CHUNK, 8)
            pltpu.sync_copy(binned_hbm.at[pl.ds(j0, CHUNK)], chunk_v)
            for kk in range(CHUNK // 16):
                sl = pl.ds(kk * 16, 16)
                v = chunk_v[sl]
                ids_v[sl] = jnp.minimum(v >> 10, NPAD - 1)
                rows_v[sl] = v & 1023
            pltpu.async_copy(x_hbm.at[ids_v], xrows_v, sem).wait()
            cc2 = jnp.minimum(cnt - k * CHUNK, CHUNK)

            def pt_body(i, carry2):
                row = _scalar_at(rows_v, i)
                rowv = jnp.zeros((16,), jnp.int32) + row
                for ccc in range(4):
                    xv = xrows_v[i, pl.ds(ccc * 16, 16)]
                    cur = plsc.load_gather(canvas_v, [featv[ccc], rowv])
                    plsc.store_scatter(
                        canvas_v, [featv[ccc], rowv], jnp.maximum(cur, xv)
                    )
                return carry2

            lax.fori_loop(0, cc2, pt_body, 0)
            return carry

        nch = (cnt + CHUNK - 1) // CHUNK
        lax.fori_loop(0, nch, chunk_body, 0)

        def wb_body(j, carry):
            pltpu.async_copy(canvas_v.at[j], out_hbm.at[b, j, t, cy4], sem).wait()
            return carry

        lax.fori_loop(0, D_OUT, wb_body, 0)


_K1A_BLK = 50176
_K1B_BLK = 6272


def _k1a_body(ptsT_ref, seg_ref):
    x = ptsT_ref[1, :]
    y = ptsT_ref[2, :]
    pcx = (x - PC_RANGE[0]) / VOXEL_SIZE[0]
    pcy = (y - PC_RANGE[1]) / VOXEL_SIZE[1]
    mask = (pcx >= 0) & (pcx < GX) & (pcy >= 0) & (pcy < GY)
    cxi = jnp.clip(jnp.floor(pcx).astype(jnp.int32), 0, GX - 1)
    cyi = jnp.clip(jnp.floor(pcy).astype(jnp.int32), 0, GY - 1)
    b = ptsT_ref[0, :].astype(jnp.int32)
    t = ptsT_ref[5, :].astype(jnp.int32)
    seg = ((b * NSWEEPS + t) * GY + cyi) * GX + cxi
    seg_ref[...] = jnp.where(mask, seg, NSEG).reshape(_K1A_BLK // 128, 128)


def _k1a_seg(ptsT):
    grid = NPAD // _K1A_BLK
    return pl.pallas_call(
        _k1a_body,
        grid=(grid,),
        in_specs=[pl.BlockSpec((6, _K1A_BLK), lambda i: (0, i))],
        out_specs=pl.BlockSpec((_K1A_BLK // 128, 128), lambda i: (i, 0)),
        out_shape=jax.ShapeDtypeStruct((NPAD // 128, 128), jnp.int32),
    )(ptsT)


def _k1b_body(pts_ref, xyzv_ref, p16_ref):
    i = pl.program_id(0)
    pts = pts_ref[...]
    x = pts[:, 1]
    y = pts[:, 2]
    z = pts[:, 3]
    inten = pts[:, 4]
    pcx = (x - PC_RANGE[0]) / VOXEL_SIZE[0]
    pcy = (y - PC_RANGE[1]) / VOXEL_SIZE[1]
    mask = (pcx >= 0) & (pcx < GX) & (pcy >= 0) & (pcy < GY)
    valid = mask.astype(jnp.float32)
    fx = jnp.floor(pcx)
    fy = jnp.floor(pcy)
    fcx = x - (fx * VOXEL_SIZE[0] + VOXEL_SIZE[0] / 2.0 + PC_RANGE[0])
    fcy = y - (fy * VOXEL_SIZE[1] + VOXEL_SIZE[1] / 2.0 + PC_RANGE[1])
    gidx = i * _K1B_BLK + lax.broadcasted_iota(jnp.int32, (_K1B_BLK,), 0)
    isreal = (gidx < N_RAW).astype(jnp.float32)
    zcol = jnp.zeros((_K1B_BLK,), jnp.float32)
    xyzv_ref[...] = jnp.stack(
        [x * valid, y * valid, z * valid, valid] + [zcol] * 12, axis=1
    )
    p16_ref[...] = jnp.stack(
        [x, y, z, inten, fcx, fcy, isreal, valid] + [zcol] * 8, axis=1
    )


def _k1b_feat(pts):
    grid = NPAD // _K1B_BLK
    return pl.pallas_call(
        _k1b_body,
        grid=(grid,),
        in_specs=[pl.BlockSpec((_K1B_BLK, 6), lambda i: (i, 0))],
        out_specs=(
            pl.BlockSpec((_K1B_BLK, 16), lambda i: (i, 0)),
            pl.BlockSpec((_K1B_BLK, 16), lambda i: (i, 0)),
        ),
        out_shape=(
            jax.ShapeDtypeStruct((NPAD, 16), jnp.float32),
            jax.ShapeDtypeStruct((NPAD, 16), jnp.float32),
        ),
    )(pts)


_K5_BLK = 6272


def _k5_body(p16_ref, meanpt_ref, feats_ref, m2_ref):
    i = pl.program_id(0)
    p16 = p16_ref[...]
    mp = meanpt_ref[...]
    valid = p16[:, 7:8]
    isreal = p16[:, 6:7]
    xyz = p16[:, 0:3]
    fcl = xyz - jnp.where(valid > 0, mp[:, 0:3], 0.0)
    zc = jnp.zeros((_K5_BLK, 6), jnp.float32)
    feats = jnp.concatenate(
        [p16[:, 0:4], fcl, p16[:, 4:6], jnp.ones_like(isreal), zc], axis=1
    )
    feats = feats * isreal
    feats_ref[...] = feats
    part = jax.lax.dot_general(
        feats, feats, (((0,), (0,)), ((), ())),
        precision=jax.lax.Precision.HIGHEST,
    )

    @pl.when(i == 0)
    def _init():
        m2_ref[...] = jnp.zeros_like(m2_ref)

    m2_ref[...] += part


def _k5_feats(p16, meanpt):
    grid = NPAD // _K5_BLK
    return pl.pallas_call(
        _k5_body,
        grid=(grid,),
        in_specs=[
            pl.BlockSpec((_K5_BLK, 16), lambda i: (i, 0)),
            pl.BlockSpec((_K5_BLK, 16), lambda i: (i, 0)),
        ],
        out_specs=(
            pl.BlockSpec((_K5_BLK, 16), lambda i: (i, 0)),
            pl.BlockSpec((16, 16), lambda i: (0, 0)),
        ),
        out_shape=(
            jax.ShapeDtypeStruct((NPAD, 16), jnp.float32),
            jax.ShapeDtypeStruct((16, 16), jnp.float32),
        ),
    )(p16, meanpt)


def _k6_body(feats_ref, m2_ref, w_ref, g_ref, bt_ref, x_ref):
    m2 = m2_ref[...]
    w16 = w_ref[...]
    mu = jax.lax.dot_general(
        m2[9:10, :] / N_RAW, w16, (((1,), (0,)), ((), ())),
        precision=jax.lax.Precision.HIGHEST,
    )
    a = jax.lax.dot_general(
        m2, w16, (((1,), (0,)), ((), ())),
        precision=jax.lax.Precision.HIGHEST,
    )
    ex2 = jnp.sum(w16 * a, axis=0, keepdims=True) / N_RAW
    var = ex2 - mu * mu
    scale = g_ref[...] / jnp.sqrt(var + BN_EPS)
    wp = w16 * scale
    bp = bt_ref[...] - mu * scale
    x = jax.lax.dot_general(
        feats_ref[...], wp, (((1,), (0,)), ((), ())),
        precision=jax.lax.Precision.HIGHEST,
    )
    x_ref[...] = jnp.maximum(x + bp, 0.0)


def _k6_matmul(feats, m2, w16, gamma2, beta2):
    grid = NPAD // _K5_BLK
    return pl.pallas_call(
        _k6_body,
        grid=(grid,),
        in_specs=[
            pl.BlockSpec((_K5_BLK, 16), lambda i: (i, 0)),
            pl.BlockSpec((16, 16), lambda i: (0, 0)),
            pl.BlockSpec((16, D_OUT), lambda i: (0, 0)),
            pl.BlockSpec((1, D_OUT), lambda i: (0, 0)),
            pl.BlockSpec((1, D_OUT), lambda i: (0, 0)),
        ],
        out_specs=pl.BlockSpec((_K5_BLK, D_OUT), lambda i: (i, 0)),
        out_shape=jax.ShapeDtypeStruct((NPAD, D_OUT), jnp.float32),
    )(feats, m2, w16, gamma2, beta2)


def kernel(points, W, gamma, beta):
    pts = jnp.pad(points, ((0, NPAD - N_RAW), (0, 0)), constant_values=1.0e4)

    if _USE_TC_PRE:
        seg = _k1a_seg(pts.T).reshape(NPAD)
        xyzv16, p16 = _k1b_feat(pts)
    else:
        vs = jnp.asarray(VOXEL_SIZE)
        pr = jnp.asarray(PC_RANGE)
        pcoords = (pts[:, 1:4] - pr[:3]) / vs
        mask = (
            (pcoords[:, 0] >= 0) & (pcoords[:, 0] < GX)
            & (pcoords[:, 1] >= 0) & (pcoords[:, 1] < GY)
        )
        valid = mask.astype(jnp.float32)
        coords = jnp.floor(pcoords)
        ci = coords.astype(jnp.int32)
        b = pts[:, 0].astype(jnp.int32)
        t = pts[:, -1].astype(jnp.int32)
        cx = jnp.clip(ci[:, 0], 0, GX - 1)
        cy = jnp.clip(ci[:, 1], 0, GY - 1)
        seg = jnp.where(mask, ((b * NSWEEPS + t) * GY + cy) * GX + cx, NSEG)
        fc = pts[:, 1:3] - (coords[:, :2] * vs[:2] + vs[:2] / 2.0 + pr[:2])
        isreal = (jnp.arange(NPAD) < N_RAW).astype(jnp.float32)
        z8 = jnp.zeros((NPAD, 8), jnp.float32)
        xyzv16 = jnp.concatenate(
            [pts[:, 1:4] * valid[:, None], valid[:, None], z8[:, :4], z8], axis=1
        )
        p16 = jnp.concatenate(
            [pts[:, 1:5], fc, isreal[:, None], valid[:, None], z8], axis=1
        )

    if _USE_SC_BIN:
        hist, rank = _k2b1_hist_rank(seg)
        offw, goff, cnts = _k2b2_offsets(hist.reshape(N_TILES, NREGP))
        binned = _k2b3_place(seg, rank, offw.reshape(-1))
    else:
        r = seg >> 10
        order = jnp.argsort(r, stable=True)
        tot = jnp.bincount(r, length=NREGP)
        tot8 = ((tot + 7) // 8) * 8
        goff = jnp.concatenate([jnp.zeros((1,), jnp.int32),
                                jnp.cumsum(tot8)[:-1].astype(jnp.int32)])
        cnts = tot.astype(jnp.int32)
        rank_in_reg = jnp.arange(NPAD) - jnp.cumsum(
            jnp.concatenate([jnp.zeros((1,), jnp.int32), tot[:-1]])
        )[r[order]]
        dest = goff[r[order]] + rank_in_reg
        packed = order.astype(jnp.int32) * 1024 + (seg[order] & 1023)
        binned = jnp.zeros((BIN_ROWS,), jnp.int32).at[dest].set(packed)

    sums = jax.ops.segment_sum(xyzv16, seg, num_segments=MEAN_ROWS)
    mean_xla = sums / jnp.maximum(sums[:, 3:4], 1.0)
    if _USE_SC_MEAN:
        mean_km = _km_means(binned, xyzv16, goff, cnts)
        mean_tbl = mean_xla + 0.0 * mean_km if _KM_SHADOW else mean_km
    else:
        mean_tbl = mean_xla

    meanpt = _k4_gather_mean(mean_tbl, seg)

    if _USE_TC_PRE:
        feats, m2 = _k5_feats(p16, meanpt)
        w16 = jnp.pad(W, ((0, 16 - 9), (0, 0)))
        x = _k6_matmul(feats, m2, w16, gamma[None, :], beta[None, :])
    else:
        valid = p16[:, 7:8]
        isreal = p16[:, 6:7]
        fcl = p16[:, 0:3] - jnp.where(valid > 0, meanpt[:, 0:3], 0.0)
        feats = jnp.concatenate(
            [p16[:, 0:4], fcl, p16[:, 4:6]], axis=1
        ) * isreal
        xr = feats @ W
        xv = xr[:N_RAW]
        mu = jnp.mean(xv, axis=0)
        var = jnp.var(xv, axis=0)
        x = (xr - mu) / jnp.sqrt(var + BN_EPS) * gamma + beta
        x = jax.nn.relu(x)

    if _USE_SC_MAX:
        out5 = _k8_scatter_max(binned, x, goff, cnts)
        return out5.reshape(BATCH, D_OUT, NSWEEPS, GY, GX)
    else:
        feat_max = jax.ops.segment_max(
            x[:N_RAW], seg[:N_RAW], num_segments=NSEG + 1
        )[:NSEG]
        cnt_seg = jnp.zeros((NSEG,), jnp.float32).at[seg[:N_RAW]].add(
            p16[:N_RAW, 7]
        ) if False else None
        occ = jax.ops.segment_sum(
            jnp.ones((N_RAW,), jnp.float32), seg[:N_RAW], num_segments=NSEG + 1
        )[:NSEG] > 0
        canvas = jnp.where(occ[:, None], feat_max, 0.0)
        canvas = canvas.reshape(BATCH, NSWEEPS, GY, GX, D_OUT)
        return jnp.transpose(canvas, (0, 4, 1, 2, 3))


# speedup vs baseline: 1.1896x; 1.1896x over previous
import functools

import jax
import jax.numpy as jnp
import numpy as np
from jax import lax
from jax.experimental import pallas as pl
from jax.experimental.pallas import tpu as pltpu
from jax.experimental.pallas import tpu_sc as plsc

VOXEL_SIZE = np.array([0.4, 0.4, 8.0], dtype=np.float32)
PC_RANGE = np.array([-51.2, -51.2, -5.0, 51.2, 51.2, 3.0], dtype=np.float32)
BATCH = 2
NSWEEPS = 2
GX = 256
GY = 256
D_OUT = 64
BN_EPS = 1e-3
N_RAW = 200000

NSEG = BATCH * NSWEEPS * GY * GX
NPAD = 200704
N_TILES = 32
PPT = NPAD // N_TILES
MEAN_ROWS = NSEG + 1024

_MESH = dict(core_axis_name="c", subcore_axis_name="s")
_SC_PARAMS = pltpu.CompilerParams(
    use_tc_tiling_on_sc=False, needs_layout_passes=False
)


@functools.partial(
    pl.kernel,
    mesh=plsc.VectorSubcoreMesh(**_MESH),
    compiler_params=_SC_PARAMS,
    out_type=jax.ShapeDtypeStruct((NPAD, 16), jnp.float32),
    scratch_types=[
        pltpu.VMEM((PPT,), jnp.int32),
        pltpu.VMEM((PPT, 16), jnp.float32),
        pltpu.SemaphoreType.DMA,
    ],
)
def _k4_gather_mean(mean_hbm, seg_hbm, out_hbm, idx_v, rows_v, sem):
    c = lax.axis_index("c")
    s = lax.axis_index("s")
    base = (c * 16 + s) * PPT
    pltpu.sync_copy(seg_hbm.at[pl.ds(base, PPT)], idx_v)
    pltpu.async_copy(mean_hbm.at[idx_v], rows_v, sem).wait()
    pltpu.sync_copy(rows_v, out_hbm.at[pl.ds(base, PPT)])


_K1A_BLK = 50176
_K1B_BLK = 6272


def _k1a_body(ptsT_ref, seg_ref):
    x = ptsT_ref[1, :]
    y = ptsT_ref[2, :]
    pcx = (x - PC_RANGE[0]) / VOXEL_SIZE[0]
    pcy = (y - PC_RANGE[1]) / VOXEL_SIZE[1]
    mask = (pcx >= 0) & (pcx < GX) & (pcy >= 0) & (pcy < GY)
    cxi = jnp.clip(jnp.floor(pcx).astype(jnp.int32), 0, GX - 1)
    cyi = jnp.clip(jnp.floor(pcy).astype(jnp.int32), 0, GY - 1)
    b = ptsT_ref[0, :].astype(jnp.int32)
    t = ptsT_ref[5, :].astype(jnp.int32)
    seg = ((b * NSWEEPS + t) * GY + cyi) * GX + cxi
    seg_ref[...] = jnp.where(mask, seg, NSEG).reshape(_K1A_BLK // 128, 128)


def _k1a_seg(ptsT):
    grid = NPAD // _K1A_BLK
    return pl.pallas_call(
        _k1a_body,
        grid=(grid,),
        in_specs=[pl.BlockSpec((6, _K1A_BLK), lambda i: (0, i))],
        out_specs=pl.BlockSpec((_K1A_BLK // 128, 128), lambda i: (i, 0)),
        out_shape=jax.ShapeDtypeStruct((NPAD // 128, 128), jnp.int32),
    )(ptsT)


def _k1b_body(pts_ref, xyzv_ref, p16_ref):
    i = pl.program_id(0)
    pts = pts_ref[...]
    x = pts[:, 1]
    y = pts[:, 2]
    z = pts[:, 3]
    inten = pts[:, 4]
    pcx = (x - PC_RANGE[0]) / VOXEL_SIZE[0]
    pcy = (y - PC_RANGE[1]) / VOXEL_SIZE[1]
    mask = (pcx >= 0) & (pcx < GX) & (pcy >= 0) & (pcy < GY)
    valid = mask.astype(jnp.float32)
    fx = jnp.floor(pcx)
    fy = jnp.floor(pcy)
    fcx = x - (fx * VOXEL_SIZE[0] + VOXEL_SIZE[0] / 2.0 + PC_RANGE[0])
    fcy = y - (fy * VOXEL_SIZE[1] + VOXEL_SIZE[1] / 2.0 + PC_RANGE[1])
    gidx = i * _K1B_BLK + lax.broadcasted_iota(jnp.int32, (_K1B_BLK,), 0)
    isreal = (gidx < N_RAW).astype(jnp.float32)
    zcol = jnp.zeros((_K1B_BLK,), jnp.float32)
    xyzv_ref[...] = jnp.stack(
        [x * valid, y * valid, z * valid, valid] + [zcol] * 12, axis=1
    )
    p16_ref[...] = jnp.stack(
        [x, y, z, inten, fcx, fcy, isreal, valid] + [zcol] * 8, axis=1
    )


def _k1b_feat(pts):
    grid = NPAD // _K1B_BLK
    return pl.pallas_call(
        _k1b_body,
        grid=(grid,),
        in_specs=[pl.BlockSpec((_K1B_BLK, 6), lambda i: (i, 0))],
        out_specs=(
            pl.BlockSpec((_K1B_BLK, 16), lambda i: (i, 0)),
            pl.BlockSpec((_K1B_BLK, 16), lambda i: (i, 0)),
        ),
        out_shape=(
            jax.ShapeDtypeStruct((NPAD, 16), jnp.float32),
            jax.ShapeDtypeStruct((NPAD, 16), jnp.float32),
        ),
    )(pts)


_K5_BLK = 6272


def _k5_body(p16_ref, meanpt_ref, feats_ref, m2_ref):
    i = pl.program_id(0)
    p16 = p16_ref[...]
    mp = meanpt_ref[...]
    valid = p16[:, 7:8]
    isreal = p16[:, 6:7]
    xyz = p16[:, 0:3]
    fcl = xyz - jnp.where(valid > 0, mp[:, 0:3], 0.0)
    zc = jnp.zeros((_K5_BLK, 6), jnp.float32)
    feats = jnp.concatenate(
        [p16[:, 0:4], fcl, p16[:, 4:6], jnp.ones_like(isreal), zc], axis=1
    )
    feats = feats * isreal
    feats_ref[...] = feats
    part = jax.lax.dot_general(
        feats, feats, (((0,), (0,)), ((), ())),
        precision=jax.lax.Precision.HIGHEST,
    )

    @pl.when(i == 0)
    def _init():
        m2_ref[...] = jnp.zeros_like(m2_ref)

    m2_ref[...] += part


def _k5_feats(p16, meanpt):
    grid = NPAD // _K5_BLK
    return pl.pallas_call(
        _k5_body,
        grid=(grid,),
        in_specs=[
            pl.BlockSpec((_K5_BLK, 16), lambda i: (i, 0)),
            pl.BlockSpec((_K5_BLK, 16), lambda i: (i, 0)),
        ],
        out_specs=(
            pl.BlockSpec((_K5_BLK, 16), lambda i: (i, 0)),
            pl.BlockSpec((16, 16), lambda i: (0, 0)),
        ),
        out_shape=(
            jax.ShapeDtypeStruct((NPAD, 16), jnp.float32),
            jax.ShapeDtypeStruct((16, 16), jnp.float32),
        ),
    )(p16, meanpt)


def _k6_body(feats_ref, m2_ref, w_ref, g_ref, bt_ref, x_ref):
    m2 = m2_ref[...]
    w16 = w_ref[...]
    mu = jax.lax.dot_general(
        m2[9:10, :] / N_RAW, w16, (((1,), (0,)), ((), ())),
        precision=jax.lax.Precision.HIGHEST,
    )
    a = jax.lax.dot_general(
        m2, w16, (((1,), (0,)), ((), ())),
        precision=jax.lax.Precision.HIGHEST,
    )
    ex2 = jnp.sum(w16 * a, axis=0, keepdims=True) / N_RAW
    var = ex2 - mu * mu
    scale = g_ref[...] / jnp.sqrt(var + BN_EPS)
    wp = w16 * scale
    bp = bt_ref[...] - mu * scale
    x = jax.lax.dot_general(
        feats_ref[...], wp, (((1,), (0,)), ((), ())),
        precision=jax.lax.Precision.HIGHEST,
    )
    x_ref[...] = jnp.maximum(x + bp, 0.0)


def _k6_matmul(feats, m2, w16, gamma2, beta2):
    grid = NPAD // _K5_BLK
    return pl.pallas_call(
        _k6_body,
        grid=(grid,),
        in_specs=[
            pl.BlockSpec((_K5_BLK, 16), lambda i: (i, 0)),
            pl.BlockSpec((16, 16), lambda i: (0, 0)),
            pl.BlockSpec((16, D_OUT), lambda i: (0, 0)),
            pl.BlockSpec((1, D_OUT), lambda i: (0, 0)),
            pl.BlockSpec((1, D_OUT), lambda i: (0, 0)),
        ],
        out_specs=pl.BlockSpec((_K5_BLK, D_OUT), lambda i: (i, 0)),
        out_shape=jax.ShapeDtypeStruct((NPAD, D_OUT), jnp.float32),
    )(feats, m2, w16, gamma2, beta2)


def kernel(points, W, gamma, beta):
    pts = jnp.pad(points, ((0, NPAD - N_RAW), (0, 0)), constant_values=1.0e4)

    seg = _k1a_seg(pts.T).reshape(NPAD)
    xyzv16, p16 = _k1b_feat(pts)

    sums = jax.ops.segment_sum(xyzv16[:, :4], seg, num_segments=MEAN_ROWS)
    mean_tbl = sums / jnp.maximum(sums[:, 3:4], 1.0)
    mean_tbl = jnp.pad(mean_tbl, ((0, 0), (0, 12)))

    meanpt = _k4_gather_mean(mean_tbl, seg)

    feats, m2 = _k5_feats(p16, meanpt)
    w16 = jnp.pad(W, ((0, 16 - 9), (0, 0)))
    x = _k6_matmul(feats, m2, w16, gamma[None, :], beta[None, :])

    feat_max = jax.ops.segment_max(
        x[:N_RAW], seg[:N_RAW], num_segments=NSEG + 1
    )[:NSEG]
    occ = sums[:NSEG, 3:4] > 0
    canvas = jnp.where(occ, feat_max, 0.0)
    canvas = canvas.reshape(BATCH, NSWEEPS, GY, GX, D_OUT)
    return jnp.transpose(canvas, (0, 4, 1, 2, 3))
